# baseline (speedup 1.0000x reference)
import jax
import jax.numpy as jnp
from jax import lax
from jax.experimental import pallas as pl
from jax.experimental.pallas import tpu as pltpu
from jax.experimental.pallas import tpu_sc as plsc

VOCAB = 1_000_000
D = 64
SCALE = 8.0

NC = 2
NS = 16
NW = NC * NS

B_TOTAL = 16384 * 50
G = 128
GROUPS_TOTAL = B_TOTAL // G
GROUPS_PER_W = GROUPS_TOTAL // NW


def _body(x_hbm, table_hbm, out_hbm, idx_v, rows_v, sem):
    wid = lax.axis_index("s") * NC + lax.axis_index("c")
    g0 = wid * GROUPS_PER_W

    pltpu.sync_copy(x_hbm.at[pl.ds(g0, GROUPS_PER_W)], idx_v)

    def step(g, _):
        pltpu.async_copy(table_hbm.at[idx_v.at[g]], rows_v, sem).wait()

        def mul(i, _):
            r = i >> 2
            c = (i & 3) * 16
            rows_v[r, pl.ds(c, 16)] = rows_v[r, pl.ds(c, 16)] * SCALE
            return 0

        lax.fori_loop(0, G * D // 16, mul, 0)

        pltpu.sync_copy(rows_v, out_hbm.at[pl.ds((g0 + g) * G, G)])
        return 0

    lax.fori_loop(0, GROUPS_PER_W, step, 0)


def kernel(x, E):
    x_flat = x.reshape(GROUPS_TOTAL, G).astype(jnp.int32)
    mesh = plsc.VectorSubcoreMesh(
        core_axis_name="c", subcore_axis_name="s", num_cores=NC, num_subcores=NS
    )
    out = pl.kernel(
        _body,
        out_type=jax.ShapeDtypeStruct((B_TOTAL, D), jnp.float32),
        mesh=mesh,
        scratch_types=[
            pltpu.VMEM((GROUPS_PER_W, G), jnp.int32),
            pltpu.VMEM((G, D), jnp.float32),
            pltpu.SemaphoreType.DMA,
        ],
        compiler_params=pltpu.CompilerParams(use_tc_tiling_on_sc=False),
    )(x_flat, E)
    return out.reshape(x.shape[0], x.shape[1], D)


# speedup vs baseline: 1.5231x; 1.5231x over previous
import jax
import jax.numpy as jnp
from jax import lax
from jax.experimental import pallas as pl
from jax.experimental.pallas import tpu as pltpu
from jax.experimental.pallas import tpu_sc as plsc

D = 64
SCALE = 8.0

NC = 2
NS = 16
NW = NC * NS

B_TOTAL = 16384 * 50
G = 128
C = 256
KG = C // G
ROWS_PER_W = B_TOTAL // NW
NCH = ROWS_PER_W // C
IDX_ROWS = ROWS_PER_W // G


def _body(x_hbm, table_hbm, out_hbm,
          idx_v, g0_v, g1_v, o0_v, o1_v,
          gsem0, gsem1, wsem0, wsem1):
    wid = lax.axis_index("s") * NC + lax.axis_index("c")
    row0 = wid * ROWS_PER_W

    gbuf = (g0_v, g1_v)
    obuf = (o0_v, o1_v)
    gsem = (gsem0, gsem1)
    wsem = (wsem0, wsem1)

    pltpu.sync_copy(x_hbm.at[pl.ds(wid * IDX_ROWS, IDX_ROWS)], idx_v)

    def fire(c, b):
        for k in range(KG):
            pltpu.async_copy(
                table_hbm.at[idx_v.at[c * KG + k]],
                gbuf[b].at[pl.ds(k * G, G)],
                gsem[b],
            )

    def wait_gather(b):
        pltpu.make_async_copy(
            out_hbm.at[pl.ds(0, C)], gbuf[b], gsem[b]
        ).wait()

    def scale(b):
        @plsc.parallel_loop(0, C, step=1, unroll=4)
        def _(r):
            for j in range(D // 16):
                obuf[b][r, pl.ds(j * 16, 16)] = (
                    gbuf[b][r, pl.ds(j * 16, 16)] * SCALE
                )

    def start_wb(c, b):
        pltpu.async_copy(obuf[b], out_hbm.at[pl.ds(row0 + c * C, C)], wsem[b])

    def wait_wb(b):
        pltpu.make_async_copy(obuf[b], out_hbm.at[pl.ds(0, C)], wsem[b]).wait()

    fire(0, 0)
    fire(1, 1)
    for b in range(2):
        wait_gather(b)
        scale(b)
        fire(2 + b, b)
        start_wb(b, b)

    def step(o, _):
        for b in range(2):
            c = 2 * o + b
            wait_gather(b)
            wait_wb(b)
            scale(b)
            fire(c + 2, b)
            start_wb(c, b)
        return 0

    lax.fori_loop(1, NCH // 2 - 1, step, 0)

    for b in range(2):
        c = NCH - 2 + b
        wait_gather(b)
        wait_wb(b)
        scale(b)
        start_wb(c, b)
    for b in range(2):
        wait_wb(b)


def kernel(x, E):
    x_flat = x.reshape(B_TOTAL // G, G).astype(jnp.int32)
    mesh = plsc.VectorSubcoreMesh(
        core_axis_name="c", subcore_axis_name="s", num_cores=NC, num_subcores=NS
    )
    out = pl.kernel(
        _body,
        out_type=jax.ShapeDtypeStruct((B_TOTAL, D), jnp.float32),
        mesh=mesh,
        scratch_types=[
            pltpu.VMEM((IDX_ROWS, G), jnp.int32),
            pltpu.VMEM((C, D), jnp.float32),
            pltpu.VMEM((C, D), jnp.float32),
            pltpu.VMEM((C, D), jnp.float32),
            pltpu.VMEM((C, D), jnp.float32),
            pltpu.SemaphoreType.DMA,
            pltpu.SemaphoreType.DMA,
            pltpu.SemaphoreType.DMA,
            pltpu.SemaphoreType.DMA,
        ],
        compiler_params=pltpu.CompilerParams(use_tc_tiling_on_sc=False),
    )(x_flat, E)
    return out.reshape(x.shape[0], x.shape[1], D)
